# baseline (speedup 1.0000x reference)
import functools

import jax
import jax.numpy as jnp
import numpy as np
from jax import lax
from jax.experimental import pallas as pl
from jax.experimental.pallas import tpu as pltpu
from jax.experimental.pallas import tpu_sc as plsc

T = 500
STRUC_N = 517
SEQ_N = 33
STRUC_MASK = 516
SEQ_MASK = 32
EPS = 1e-10
_NEG = np.int32(-(2 ** 31))
_BIG = np.int32(2 ** 30)
_LANES = 128
_CHUNK = 40

def _i32(x):
    return int(np.uint32(x & 0xFFFFFFFF).astype(np.int32))

_KS_S = (1832780943, 270669613)
_KS_Q = (64467757, _i32(2916123636))

_SC_ROWS = 8192
_NW = 32
_RW = _SC_ROWS // _NW
_GPI = 4


def _sc_keyset(k1, k2):
    ks2 = _i32(k1 ^ k2 ^ 0x1BD11BDA)
    return (_i32(k1), _i32(k2), ks2)


def _sc_tf(x1, ks):
    i32 = np.int32
    x0 = jnp.zeros_like(x1) + i32(ks[0])
    x1 = x1 + i32(ks[1])
    rot = (13, 15, 26, 6, 17, 29, 16, 24)
    rounds = (rot[0:4], rot[4:8], rot[0:4], rot[4:8], rot[0:4])
    for i, chunk in enumerate(rounds):
        for r in chunk:
            x0 = x0 + x1
            x1 = lax.shift_left(x1, i32(r)) | lax.shift_right_logical(
                x1, i32(32 - r))
            x1 = x0 ^ x1
        x0 = x0 + i32(ks[(i + 1) % 3])
        x1 = x1 + i32(_i32(ks[(i + 2) % 3] + (i + 1)))
    return x0 ^ x1 ^ i32(-(2 ** 31))


def _sc_struct(x_sc, row0):
    ks = _sc_keyset(*_KS_S)
    mesh = plsc.VectorSubcoreMesh(core_axis_name="c", subcore_axis_name="s")

    @functools.partial(
        pl.kernel, mesh=mesh,
        out_type=jax.ShapeDtypeStruct((4 * _SC_ROWS,), jnp.int32),
        scratch_types=[
            pltpu.VMEM((_RW,), jnp.int32),
            pltpu.VMEM((_RW,), jnp.int32),
            pltpu.VMEM((_RW,), jnp.int32),
            pltpu.VMEM((_RW,), jnp.int32),
            pltpu.VMEM((_RW,), jnp.int32),
        ],
    )
    def sck(x_hbm, out_hbm, xv, sx0, smk, sav, sai):
        wid = lax.axis_index("s") * 2 + lax.axis_index("c")
        base = wid * _RW
        pltpu.sync_copy(x_hbm.at[pl.ds(base, _RW)], xv)
        iota = lax.iota(jnp.int32, 16)
        for q in range(_RW // (16 * _GPI)):
            off = q * 16 * _GPI
            x0v, ib = [], []
            for j in range(_GPI):
                xj = xv[pl.ds(off + j * 16, 16)]
                rowv = row0 + base + off + j * 16 + iota
                ibj = rowv * np.int32(STRUC_N)
                x0v.append(xj)
                ib.append(ibj)
                sx0[pl.ds(off + j * 16, 16)] = _sc_tf(ibj + xj, ks)
                smk[pl.ds(off + j * 16, 16)] = _sc_tf(
                    ibj + np.int32(STRUC_MASK), ks)

            def cls_body(c, carry):
                avs, ais = carry
                navs, nais = [], []
                for j in range(_GPI):
                    b = _sc_tf(ib[j] + c, ks)
                    be = jnp.where(x0v[j] == c, _NEG, b)
                    upd = be > avs[j]
                    navs.append(jnp.maximum(avs[j], be))
                    nais.append(jnp.where(upd, c, ais[j]))
                return tuple(navs), tuple(nais)

            init = (tuple(jnp.full((16,), _NEG, jnp.int32)
                          for _ in range(_GPI)),
                    tuple(jnp.full((16,), _BIG, jnp.int32)
                          for _ in range(_GPI)))
            avs, ais = lax.fori_loop(0, STRUC_N - 1, cls_body, init)
            for j in range(_GPI):
                sav[pl.ds(off + j * 16, 16)] = avs[j]
                sai[pl.ds(off + j * 16, 16)] = ais[j]
        pltpu.sync_copy(sx0, out_hbm.at[pl.ds(0 * _SC_ROWS + base, _RW)])
        pltpu.sync_copy(smk, out_hbm.at[pl.ds(1 * _SC_ROWS + base, _RW)])
        pltpu.sync_copy(sav, out_hbm.at[pl.ds(2 * _SC_ROWS + base, _RW)])
        pltpu.sync_copy(sai, out_hbm.at[pl.ds(3 * _SC_ROWS + base, _RW)])

    return sck(x_sc)


def _threefry_biased(k1, k2, x1):
    ks2 = k1 ^ k2 ^ jnp.uint32(0x1BD11BDA)
    ks = (k1, k2, ks2)
    x0 = jnp.zeros_like(x1) + k1
    x1 = x1 + k2
    rot = (13, 15, 26, 6, 17, 29, 16, 24)
    rounds = (rot[0:4], rot[4:8], rot[0:4], rot[4:8], rot[0:4])
    for i, chunk in enumerate(rounds):
        for r in chunk:
            x0 = x0 + x1
            x1 = (x1 << jnp.uint32(r)) | (x1 >> jnp.uint32(32 - r))
            x1 = x0 ^ x1
        x0 = x0 + ks[(i + 1) % 3]
        x1 = x1 + ks[(i + 2) % 3] + jnp.uint32(i + 1)
    return jax.lax.bitcast_convert_type(x0 ^ x1 ^ jnp.uint32(0x80000000),
                                        jnp.int32)


def _combine(av, ai, bv, bi):
    take = (bv > av) | ((bv == av) & (bi < ai))
    return jnp.maximum(av, bv), jnp.where(take, bi, ai)


def _tail_scan(k1, k2, ibase, c_loc, x0, n_cls, n_pad):
    acc_v = jnp.full((_CHUNK, _LANES), _NEG, jnp.int32)
    acc_i = jnp.full((_CHUNK, _LANES), _BIG, jnp.int32)
    for c0 in range(0, n_pad, _CHUNK):
        biased = _threefry_biased(k1, k2, (ibase + c0).astype(jnp.uint32))
        excl = c_loc == (x0 - c0)
        thr = n_cls - 1 - c0
        if thr < _CHUNK:
            excl = excl | (c_loc >= thr)
        b_eps = jnp.where(excl, _NEG, biased)
        upd = b_eps > acc_v
        acc_v = jnp.maximum(acc_v, b_eps)
        acc_i = jnp.where(upd, c_loc + c0, acc_i)
    n = _CHUNK
    while n > 1:
        h = n // 2
        mv, mi = _combine(acc_v[:h], acc_i[:h], acc_v[h:2 * h],
                          acc_i[h:2 * h])
        if n % 2:
            mv = jnp.concatenate([mv, acc_v[2 * h:n]], axis=0)
            mi = jnp.concatenate([mi, acc_i[2 * h:n]], axis=0)
        acc_v, acc_i = mv, mi
        n = h + (n % 2)
    return acc_v, acc_i


def _both_body(keys_ref, xs_ref, xq_ref, a_ref, out_ref, *, tc_progs):
    p = pl.program_id(0)
    u32 = lambda v: jax.lax.bitcast_convert_type(v, jnp.uint32)
    k1s, k2s = u32(keys_ref[0]), u32(keys_ref[1])
    k1q, k2q = u32(keys_ref[2]), u32(keys_ref[3])
    xs = xs_ref[0]
    xq = xq_ref[0]
    lane1 = jax.lax.broadcasted_iota(jnp.int32, (1, _LANES), 1)
    row1 = p * _LANES + lane1
    ibs1 = row1 * STRUC_N
    ibq1 = row1 * SEQ_N

    cand_i = jnp.concatenate(
        [ibs1 + xs, ibs1 + STRUC_MASK, ibq1 + xq, ibq1 + SEQ_MASK,
         jnp.zeros((4, _LANES), jnp.int32)], axis=0)
    srow = jax.lax.broadcasted_iota(jnp.int32, (8, 1), 0) < 2
    ck1 = jnp.where(srow, k1s, k1q)
    ck2 = jnp.where(srow, k2s, k2q)
    cand_b = _threefry_biased(ck1, ck2, cand_i.astype(jnp.uint32))

    c_loc = jax.lax.broadcasted_iota(jnp.int32, (_CHUNK, _LANES), 0)
    lane = jax.lax.broadcasted_iota(jnp.int32, (_CHUNK, _LANES), 1)
    row = p * _LANES + lane
    a = a_ref[0]
    unb = lambda b: jax.lax.bitcast_convert_type(b, jnp.uint32) ^ jnp.uint32(
        0x80000000)

    @pl.when(p < tc_progs)
    def _struct():
        sv, si = _tail_scan(k1s, k2s, row * STRUC_N + c_loc, c_loc, xs,
                            STRUC_N, 520)
        tok_s = _finish(unb(cand_b[0:1, :]), unb(cand_b[1:2, :]), unb(sv),
                        si, xs, a, STRUC_MASK)
        out_ref[0, 0:1, :] = tok_s

    qv, qi = _tail_scan(k1q, k2q, row * SEQ_N + c_loc, c_loc, xq,
                        SEQ_N, 40)
    tok_q = _finish(unb(cand_b[2:3, :]), unb(cand_b[3:4, :]), unb(qv), qi,
                    xq, a, SEQ_MASK)
    out_ref[0, 1:2, :] = tok_q
    out_ref[0, 2:8, :] = jnp.zeros((6, _LANES), jnp.int32)


def _gumbel_from_bits(bits):
    tiny = jnp.float32(jnp.finfo(jnp.float32).tiny)
    fb = (bits >> jnp.uint32(9)) | jnp.uint32(0x3F800000)
    floats = jax.lax.bitcast_convert_type(fb, jnp.float32) - jnp.float32(1.0)
    u = jnp.maximum(tiny, floats * (jnp.float32(1.0) - tiny) + tiny)
    return -jnp.log(-jnp.log(u))


def _finish(b_x0, b_mask, b_eps, i_eps, x_flat, a_flat, mask_idx):
    eq = x_flat == mask_idx
    one_minus_a = jnp.float32(1.0) - a_flat
    p_x0 = a_flat + jnp.where(eq, one_minus_a, jnp.float32(0.0))
    p_m = jnp.where(eq, a_flat + one_minus_a, one_minus_a)
    v1 = _gumbel_from_bits(b_x0) + jnp.log(p_x0 + EPS)
    v2 = _gumbel_from_bits(b_mask) + jnp.log(p_m + EPS)
    v3 = _gumbel_from_bits(b_eps) + jnp.log(jnp.float32(0.0) + EPS)
    i1 = x_flat
    i2 = jnp.full_like(x_flat, mask_idx)
    best_v, best_i = v1, i1
    upd = (v2 > best_v) | ((v2 == best_v) & (i2 < best_i))
    best_v = jnp.where(upd, v2, best_v)
    best_i = jnp.where(upd, i2, best_i)
    upd = (v3 > best_v) | ((v3 == best_v) & (i_eps < best_i))
    best_i = jnp.where(upd, i_eps, best_i)
    return best_i


def kernel(structure, sequence, t):
    t_idx = jnp.arange(T + 1, dtype=jnp.float32)
    beta = 1.0 / (T - t_idx + 1.0)
    alpha = jnp.cumprod(1.0 - beta)
    key = jax.random.key(42)
    ks, kq = jax.random.split(key)
    keys = jnp.concatenate([jax.random.key_data(ks),
                            jax.random.key_data(kq)]).astype(jnp.int32)
    B, L = structure.shape
    rows = B * L
    grid = rows // _LANES
    tc_rows = rows - _SC_ROWS
    tc_progs = tc_rows // _LANES
    xs = structure.reshape(grid, 1, _LANES).astype(jnp.int32)
    xq = sequence.reshape(grid, 1, _LANES).astype(jnp.int32)
    a_flat = jnp.repeat(alpha[t], L)
    a_in = a_flat.reshape(grid, 1, _LANES)
    x_struct_flat = structure.reshape(rows).astype(jnp.int32)
    sc_out = _sc_struct(x_struct_flat[tc_rows:], tc_rows)
    out = pl.pallas_call(
        functools.partial(_both_body, tc_progs=tc_progs),
        grid=(grid,),
        in_specs=[
            pl.BlockSpec(memory_space=pltpu.SMEM),
            pl.BlockSpec((1, 1, _LANES), lambda p: (p, 0, 0)),
            pl.BlockSpec((1, 1, _LANES), lambda p: (p, 0, 0)),
            pl.BlockSpec((1, 1, _LANES), lambda p: (p, 0, 0)),
        ],
        out_specs=pl.BlockSpec((1, 8, _LANES), lambda p: (p, 0, 0)),
        out_shape=jax.ShapeDtypeStruct((grid, 8, _LANES), jnp.int32),
        compiler_params=pltpu.CompilerParams(
            dimension_semantics=("parallel",)),
    )(keys, xs, xq, a_in)
    unbias = lambda b: jax.lax.bitcast_convert_type(b, jnp.uint32) ^ jnp.uint32(
        0x80000000)
    tok_sc = _finish(unbias(sc_out[0:_SC_ROWS]),
                     unbias(sc_out[_SC_ROWS:2 * _SC_ROWS]),
                     unbias(sc_out[2 * _SC_ROWS:3 * _SC_ROWS]),
                     sc_out[3 * _SC_ROWS:],
                     x_struct_flat[tc_rows:], a_flat[tc_rows:], STRUC_MASK)
    tok_s = jnp.concatenate(
        [out[:tc_progs, 0, :].reshape(tc_rows), tok_sc]).reshape(B, L)
    return (tok_s, out[:, 1, :].reshape(B, L), t)


# speedup vs baseline: 1.0900x; 1.0900x over previous
import functools

import jax
import jax.numpy as jnp
import numpy as np
from jax import lax
from jax.experimental import pallas as pl
from jax.experimental.pallas import tpu as pltpu
from jax.experimental.pallas import tpu_sc as plsc

T = 500
STRUC_N = 517
SEQ_N = 33
STRUC_MASK = 516
SEQ_MASK = 32
EPS = 1e-10
_NEG = np.int32(-(2 ** 31))
_BIG = np.int32(2 ** 30)
_LANES = 128
_CHUNK = 40

def _i32(x):
    return int(np.uint32(x & 0xFFFFFFFF).astype(np.int32))

_KS_S = (1832780943, 270669613)
_KS_Q = (64467757, _i32(2916123636))

_SC_ROWS = 16384
_NW = 32
_RW = _SC_ROWS // _NW
_GPI = 4


def _sc_keyset(k1, k2):
    ks2 = _i32(k1 ^ k2 ^ 0x1BD11BDA)
    return (_i32(k1), _i32(k2), ks2)


def _sc_tf(x1, ks):
    i32 = np.int32
    x0 = jnp.zeros_like(x1) + i32(ks[0])
    x1 = x1 + i32(ks[1])
    rot = (13, 15, 26, 6, 17, 29, 16, 24)
    rounds = (rot[0:4], rot[4:8], rot[0:4], rot[4:8], rot[0:4])
    for i, chunk in enumerate(rounds):
        for r in chunk:
            x0 = x0 + x1
            x1 = lax.shift_left(x1, i32(r)) | lax.shift_right_logical(
                x1, i32(32 - r))
            x1 = x0 ^ x1
        x0 = x0 + i32(ks[(i + 1) % 3])
        x1 = x1 + i32(_i32(ks[(i + 2) % 3] + (i + 1)))
    return x0 ^ x1 ^ i32(-(2 ** 31))


def _sc_struct(x_sc, row0):
    ks = _sc_keyset(*_KS_S)
    mesh = plsc.VectorSubcoreMesh(core_axis_name="c", subcore_axis_name="s")

    @functools.partial(
        pl.kernel, mesh=mesh,
        out_type=jax.ShapeDtypeStruct((4 * _SC_ROWS,), jnp.int32),
        scratch_types=[
            pltpu.VMEM((_RW,), jnp.int32),
            pltpu.VMEM((_RW,), jnp.int32),
            pltpu.VMEM((_RW,), jnp.int32),
            pltpu.VMEM((_RW,), jnp.int32),
            pltpu.VMEM((_RW,), jnp.int32),
        ],
    )
    def sck(x_hbm, out_hbm, xv, sx0, smk, sav, sai):
        wid = lax.axis_index("s") * 2 + lax.axis_index("c")
        base = wid * _RW
        pltpu.sync_copy(x_hbm.at[pl.ds(base, _RW)], xv)
        iota = lax.iota(jnp.int32, 16)
        for q in range(_RW // (16 * _GPI)):
            off = q * 16 * _GPI
            x0v, ib = [], []
            for j in range(_GPI):
                xj = xv[pl.ds(off + j * 16, 16)]
                rowv = row0 + base + off + j * 16 + iota
                ibj = rowv * np.int32(STRUC_N)
                x0v.append(xj)
                ib.append(ibj)
                sx0[pl.ds(off + j * 16, 16)] = _sc_tf(ibj + xj, ks)
                smk[pl.ds(off + j * 16, 16)] = _sc_tf(
                    ibj + np.int32(STRUC_MASK), ks)

            def cls_body(c, carry):
                avs, ais = carry
                navs, nais = [], []
                for j in range(_GPI):
                    b = _sc_tf(ib[j] + c, ks)
                    be = jnp.where(x0v[j] == c, _NEG, b)
                    upd = be > avs[j]
                    navs.append(jnp.maximum(avs[j], be))
                    nais.append(jnp.where(upd, c, ais[j]))
                return tuple(navs), tuple(nais)

            init = (tuple(jnp.full((16,), _NEG, jnp.int32)
                          for _ in range(_GPI)),
                    tuple(jnp.full((16,), _BIG, jnp.int32)
                          for _ in range(_GPI)))
            avs, ais = lax.fori_loop(0, STRUC_N - 1, cls_body, init)
            for j in range(_GPI):
                sav[pl.ds(off + j * 16, 16)] = avs[j]
                sai[pl.ds(off + j * 16, 16)] = ais[j]
        pltpu.sync_copy(sx0, out_hbm.at[pl.ds(0 * _SC_ROWS + base, _RW)])
        pltpu.sync_copy(smk, out_hbm.at[pl.ds(1 * _SC_ROWS + base, _RW)])
        pltpu.sync_copy(sav, out_hbm.at[pl.ds(2 * _SC_ROWS + base, _RW)])
        pltpu.sync_copy(sai, out_hbm.at[pl.ds(3 * _SC_ROWS + base, _RW)])

    return sck(x_sc)


def _threefry_biased(k1, k2, x1):
    ks2 = k1 ^ k2 ^ jnp.uint32(0x1BD11BDA)
    ks = (k1, k2, ks2)
    x0 = jnp.zeros_like(x1) + k1
    x1 = x1 + k2
    rot = (13, 15, 26, 6, 17, 29, 16, 24)
    rounds = (rot[0:4], rot[4:8], rot[0:4], rot[4:8], rot[0:4])
    for i, chunk in enumerate(rounds):
        for r in chunk:
            x0 = x0 + x1
            x1 = (x1 << jnp.uint32(r)) | (x1 >> jnp.uint32(32 - r))
            x1 = x0 ^ x1
        x0 = x0 + ks[(i + 1) % 3]
        x1 = x1 + ks[(i + 2) % 3] + jnp.uint32(i + 1)
    return jax.lax.bitcast_convert_type(x0 ^ x1 ^ jnp.uint32(0x80000000),
                                        jnp.int32)


def _combine(av, ai, bv, bi):
    take = (bv > av) | ((bv == av) & (bi < ai))
    return jnp.maximum(av, bv), jnp.where(take, bi, ai)


def _tail_scan(k1, k2, ibase, c_loc, x0, n_cls, n_pad):
    acc_v = jnp.full((_CHUNK, _LANES), _NEG, jnp.int32)
    acc_i = jnp.full((_CHUNK, _LANES), _BIG, jnp.int32)
    for c0 in range(0, n_pad, _CHUNK):
        biased = _threefry_biased(k1, k2, (ibase + c0).astype(jnp.uint32))
        excl = c_loc == (x0 - c0)
        thr = n_cls - 1 - c0
        if thr < _CHUNK:
            excl = excl | (c_loc >= thr)
        b_eps = jnp.where(excl, _NEG, biased)
        upd = b_eps > acc_v
        acc_v = jnp.maximum(acc_v, b_eps)
        acc_i = jnp.where(upd, c_loc + c0, acc_i)
    n = _CHUNK
    while n > 1:
        h = n // 2
        mv, mi = _combine(acc_v[:h], acc_i[:h], acc_v[h:2 * h],
                          acc_i[h:2 * h])
        if n % 2:
            mv = jnp.concatenate([mv, acc_v[2 * h:n]], axis=0)
            mi = jnp.concatenate([mi, acc_i[2 * h:n]], axis=0)
        acc_v, acc_i = mv, mi
        n = h + (n % 2)
    return acc_v, acc_i


def _both_body(keys_ref, xs_ref, xq_ref, a_ref, out_ref, *, tc_progs):
    p = pl.program_id(0)
    u32 = lambda v: jax.lax.bitcast_convert_type(v, jnp.uint32)
    k1s, k2s = u32(keys_ref[0]), u32(keys_ref[1])
    k1q, k2q = u32(keys_ref[2]), u32(keys_ref[3])
    xs = xs_ref[0]
    xq = xq_ref[0]
    lane1 = jax.lax.broadcasted_iota(jnp.int32, (1, _LANES), 1)
    row1 = p * _LANES + lane1
    ibs1 = row1 * STRUC_N
    ibq1 = row1 * SEQ_N

    cand_i = jnp.concatenate(
        [ibs1 + xs, ibs1 + STRUC_MASK, ibq1 + xq, ibq1 + SEQ_MASK,
         jnp.zeros((4, _LANES), jnp.int32)], axis=0)
    srow = jax.lax.broadcasted_iota(jnp.int32, (8, 1), 0) < 2
    ck1 = jnp.where(srow, k1s, k1q)
    ck2 = jnp.where(srow, k2s, k2q)
    cand_b = _threefry_biased(ck1, ck2, cand_i.astype(jnp.uint32))

    c_loc = jax.lax.broadcasted_iota(jnp.int32, (_CHUNK, _LANES), 0)
    lane = jax.lax.broadcasted_iota(jnp.int32, (_CHUNK, _LANES), 1)
    row = p * _LANES + lane
    a = a_ref[0]
    unb = lambda b: jax.lax.bitcast_convert_type(b, jnp.uint32) ^ jnp.uint32(
        0x80000000)

    @pl.when(p < tc_progs)
    def _struct():
        sv, si = _tail_scan(k1s, k2s, row * STRUC_N + c_loc, c_loc, xs,
                            STRUC_N, 520)
        tok_s = _finish(unb(cand_b[0:1, :]), unb(cand_b[1:2, :]), unb(sv),
                        si, xs, a, STRUC_MASK)
        out_ref[0, 0:1, :] = tok_s

    qv, qi = _tail_scan(k1q, k2q, row * SEQ_N + c_loc, c_loc, xq,
                        SEQ_N, 40)
    tok_q = _finish(unb(cand_b[2:3, :]), unb(cand_b[3:4, :]), unb(qv), qi,
                    xq, a, SEQ_MASK)
    out_ref[0, 1:2, :] = tok_q
    out_ref[0, 2:8, :] = jnp.zeros((6, _LANES), jnp.int32)


def _gumbel_from_bits(bits):
    tiny = jnp.float32(jnp.finfo(jnp.float32).tiny)
    fb = (bits >> jnp.uint32(9)) | jnp.uint32(0x3F800000)
    floats = jax.lax.bitcast_convert_type(fb, jnp.float32) - jnp.float32(1.0)
    u = jnp.maximum(tiny, floats * (jnp.float32(1.0) - tiny) + tiny)
    return -jnp.log(-jnp.log(u))


def _finish(b_x0, b_mask, b_eps, i_eps, x_flat, a_flat, mask_idx):
    eq = x_flat == mask_idx
    one_minus_a = jnp.float32(1.0) - a_flat
    p_x0 = a_flat + jnp.where(eq, one_minus_a, jnp.float32(0.0))
    p_m = jnp.where(eq, a_flat + one_minus_a, one_minus_a)
    v1 = _gumbel_from_bits(b_x0) + jnp.log(p_x0 + EPS)
    v2 = _gumbel_from_bits(b_mask) + jnp.log(p_m + EPS)
    v3 = _gumbel_from_bits(b_eps) + jnp.log(jnp.float32(0.0) + EPS)
    i1 = x_flat
    i2 = jnp.full_like(x_flat, mask_idx)
    best_v, best_i = v1, i1
    upd = (v2 > best_v) | ((v2 == best_v) & (i2 < best_i))
    best_v = jnp.where(upd, v2, best_v)
    best_i = jnp.where(upd, i2, best_i)
    upd = (v3 > best_v) | ((v3 == best_v) & (i_eps < best_i))
    best_i = jnp.where(upd, i_eps, best_i)
    return best_i


def kernel(structure, sequence, t):
    t_idx = jnp.arange(T + 1, dtype=jnp.float32)
    beta = 1.0 / (T - t_idx + 1.0)
    alpha = jnp.cumprod(1.0 - beta)
    key = jax.random.key(42)
    ks, kq = jax.random.split(key)
    keys = jnp.concatenate([jax.random.key_data(ks),
                            jax.random.key_data(kq)]).astype(jnp.int32)
    B, L = structure.shape
    rows = B * L
    grid = rows // _LANES
    tc_rows = rows - _SC_ROWS
    tc_progs = tc_rows // _LANES
    xs = structure.reshape(grid, 1, _LANES).astype(jnp.int32)
    xq = sequence.reshape(grid, 1, _LANES).astype(jnp.int32)
    a_flat = jnp.repeat(alpha[t], L)
    a_in = a_flat.reshape(grid, 1, _LANES)
    x_struct_flat = structure.reshape(rows).astype(jnp.int32)
    sc_out = _sc_struct(x_struct_flat[tc_rows:], tc_rows)
    out = pl.pallas_call(
        functools.partial(_both_body, tc_progs=tc_progs),
        grid=(grid,),
        in_specs=[
            pl.BlockSpec(memory_space=pltpu.SMEM),
            pl.BlockSpec((1, 1, _LANES), lambda p: (p, 0, 0)),
            pl.BlockSpec((1, 1, _LANES), lambda p: (p, 0, 0)),
            pl.BlockSpec((1, 1, _LANES), lambda p: (p, 0, 0)),
        ],
        out_specs=pl.BlockSpec((1, 8, _LANES), lambda p: (p, 0, 0)),
        out_shape=jax.ShapeDtypeStruct((grid, 8, _LANES), jnp.int32),
        compiler_params=pltpu.CompilerParams(
            dimension_semantics=("parallel",)),
    )(keys, xs, xq, a_in)
    unbias = lambda b: jax.lax.bitcast_convert_type(b, jnp.uint32) ^ jnp.uint32(
        0x80000000)
    tok_sc = _finish(unbias(sc_out[0:_SC_ROWS]),
                     unbias(sc_out[_SC_ROWS:2 * _SC_ROWS]),
                     unbias(sc_out[2 * _SC_ROWS:3 * _SC_ROWS]),
                     sc_out[3 * _SC_ROWS:],
                     x_struct_flat[tc_rows:], a_flat[tc_rows:], STRUC_MASK)
    tok_s = jnp.concatenate(
        [out[:tc_progs, 0, :].reshape(tc_rows), tok_sc]).reshape(B, L)
    return (tok_s, out[:, 1, :].reshape(B, L), t)


# speedup vs baseline: 1.0900x; 1.0000x over previous
import functools

import jax
import jax.numpy as jnp
import numpy as np
from jax import lax
from jax.experimental import pallas as pl
from jax.experimental.pallas import tpu as pltpu
from jax.experimental.pallas import tpu_sc as plsc

T = 500
STRUC_N = 517
SEQ_N = 33
STRUC_MASK = 516
SEQ_MASK = 32
EPS = 1e-10
_NEG = np.int32(-(2 ** 31))
_BIG = np.int32(2 ** 30)
_LANES = 128
_CHUNK = 40

def _i32(x):
    return int(np.uint32(x & 0xFFFFFFFF).astype(np.int32))

_KS_S = (1832780943, 270669613)
_KS_Q = (64467757, _i32(2916123636))

_SC_ROWS = 16384
_NW = 32
_RW = _SC_ROWS // _NW
_GPI = 8


def _sc_keyset(k1, k2):
    ks2 = _i32(k1 ^ k2 ^ 0x1BD11BDA)
    return (_i32(k1), _i32(k2), ks2)


def _sc_tf(x1, ks):
    i32 = np.int32
    x0 = jnp.zeros_like(x1) + i32(ks[0])
    x1 = x1 + i32(ks[1])
    rot = (13, 15, 26, 6, 17, 29, 16, 24)
    rounds = (rot[0:4], rot[4:8], rot[0:4], rot[4:8], rot[0:4])
    for i, chunk in enumerate(rounds):
        for r in chunk:
            x0 = x0 + x1
            x1 = lax.shift_left(x1, i32(r)) | lax.shift_right_logical(
                x1, i32(32 - r))
            x1 = x0 ^ x1
        x0 = x0 + i32(ks[(i + 1) % 3])
        x1 = x1 + i32(_i32(ks[(i + 2) % 3] + (i + 1)))
    return x0 ^ x1 ^ i32(-(2 ** 31))


def _sc_struct(x_sc, row0):
    ks = _sc_keyset(*_KS_S)
    mesh = plsc.VectorSubcoreMesh(core_axis_name="c", subcore_axis_name="s")

    @functools.partial(
        pl.kernel, mesh=mesh,
        out_type=jax.ShapeDtypeStruct((4 * _SC_ROWS,), jnp.int32),
        scratch_types=[
            pltpu.VMEM((_RW,), jnp.int32),
            pltpu.VMEM((_RW,), jnp.int32),
            pltpu.VMEM((_RW,), jnp.int32),
            pltpu.VMEM((_RW,), jnp.int32),
            pltpu.VMEM((_RW,), jnp.int32),
        ],
    )
    def sck(x_hbm, out_hbm, xv, sx0, smk, sav, sai):
        wid = lax.axis_index("s") * 2 + lax.axis_index("c")
        base = wid * _RW
        pltpu.sync_copy(x_hbm.at[pl.ds(base, _RW)], xv)
        iota = lax.iota(jnp.int32, 16)
        for q in range(_RW // (16 * _GPI)):
            off = q * 16 * _GPI
            x0v, ib = [], []
            for j in range(_GPI):
                xj = xv[pl.ds(off + j * 16, 16)]
                rowv = row0 + base + off + j * 16 + iota
                ibj = rowv * np.int32(STRUC_N)
                x0v.append(xj)
                ib.append(ibj)
                sx0[pl.ds(off + j * 16, 16)] = _sc_tf(ibj + xj, ks)
                smk[pl.ds(off + j * 16, 16)] = _sc_tf(
                    ibj + np.int32(STRUC_MASK), ks)

            def cls_body(c, carry):
                avs, ais = carry
                navs, nais = [], []
                for j in range(_GPI):
                    b = _sc_tf(ib[j] + c, ks)
                    be = jnp.where(x0v[j] == c, _NEG, b)
                    upd = be > avs[j]
                    navs.append(jnp.maximum(avs[j], be))
                    nais.append(jnp.where(upd, c, ais[j]))
                return tuple(navs), tuple(nais)

            init = (tuple(jnp.full((16,), _NEG, jnp.int32)
                          for _ in range(_GPI)),
                    tuple(jnp.full((16,), _BIG, jnp.int32)
                          for _ in range(_GPI)))
            avs, ais = lax.fori_loop(0, STRUC_N - 1, cls_body, init)
            for j in range(_GPI):
                sav[pl.ds(off + j * 16, 16)] = avs[j]
                sai[pl.ds(off + j * 16, 16)] = ais[j]
        pltpu.sync_copy(sx0, out_hbm.at[pl.ds(0 * _SC_ROWS + base, _RW)])
        pltpu.sync_copy(smk, out_hbm.at[pl.ds(1 * _SC_ROWS + base, _RW)])
        pltpu.sync_copy(sav, out_hbm.at[pl.ds(2 * _SC_ROWS + base, _RW)])
        pltpu.sync_copy(sai, out_hbm.at[pl.ds(3 * _SC_ROWS + base, _RW)])

    return sck(x_sc)


def _threefry_biased(k1, k2, x1):
    ks2 = k1 ^ k2 ^ jnp.uint32(0x1BD11BDA)
    ks = (k1, k2, ks2)
    x0 = jnp.zeros_like(x1) + k1
    x1 = x1 + k2
    rot = (13, 15, 26, 6, 17, 29, 16, 24)
    rounds = (rot[0:4], rot[4:8], rot[0:4], rot[4:8], rot[0:4])
    for i, chunk in enumerate(rounds):
        for r in chunk:
            x0 = x0 + x1
            x1 = (x1 << jnp.uint32(r)) | (x1 >> jnp.uint32(32 - r))
            x1 = x0 ^ x1
        x0 = x0 + ks[(i + 1) % 3]
        x1 = x1 + ks[(i + 2) % 3] + jnp.uint32(i + 1)
    return jax.lax.bitcast_convert_type(x0 ^ x1 ^ jnp.uint32(0x80000000),
                                        jnp.int32)


def _combine(av, ai, bv, bi):
    take = (bv > av) | ((bv == av) & (bi < ai))
    return jnp.maximum(av, bv), jnp.where(take, bi, ai)


def _tail_scan(k1, k2, ibase, c_loc, x0, n_cls, n_pad):
    acc_v = jnp.full((_CHUNK, _LANES), _NEG, jnp.int32)
    acc_i = jnp.full((_CHUNK, _LANES), _BIG, jnp.int32)
    for c0 in range(0, n_pad, _CHUNK):
        biased = _threefry_biased(k1, k2, (ibase + c0).astype(jnp.uint32))
        excl = c_loc == (x0 - c0)
        thr = n_cls - 1 - c0
        if thr < _CHUNK:
            excl = excl | (c_loc >= thr)
        b_eps = jnp.where(excl, _NEG, biased)
        upd = b_eps > acc_v
        acc_v = jnp.maximum(acc_v, b_eps)
        acc_i = jnp.where(upd, c_loc + c0, acc_i)
    n = _CHUNK
    while n > 1:
        h = n // 2
        mv, mi = _combine(acc_v[:h], acc_i[:h], acc_v[h:2 * h],
                          acc_i[h:2 * h])
        if n % 2:
            mv = jnp.concatenate([mv, acc_v[2 * h:n]], axis=0)
            mi = jnp.concatenate([mi, acc_i[2 * h:n]], axis=0)
        acc_v, acc_i = mv, mi
        n = h + (n % 2)
    return acc_v, acc_i


def _both_body(keys_ref, xs_ref, xq_ref, a_ref, out_ref, *, tc_progs):
    p = pl.program_id(0)
    u32 = lambda v: jax.lax.bitcast_convert_type(v, jnp.uint32)
    k1s, k2s = u32(keys_ref[0]), u32(keys_ref[1])
    k1q, k2q = u32(keys_ref[2]), u32(keys_ref[3])
    xs = xs_ref[0]
    xq = xq_ref[0]
    lane1 = jax.lax.broadcasted_iota(jnp.int32, (1, _LANES), 1)
    row1 = p * _LANES + lane1
    ibs1 = row1 * STRUC_N
    ibq1 = row1 * SEQ_N

    cand_i = jnp.concatenate(
        [ibs1 + xs, ibs1 + STRUC_MASK, ibq1 + xq, ibq1 + SEQ_MASK,
         jnp.zeros((4, _LANES), jnp.int32)], axis=0)
    srow = jax.lax.broadcasted_iota(jnp.int32, (8, 1), 0) < 2
    ck1 = jnp.where(srow, k1s, k1q)
    ck2 = jnp.where(srow, k2s, k2q)
    cand_b = _threefry_biased(ck1, ck2, cand_i.astype(jnp.uint32))

    c_loc = jax.lax.broadcasted_iota(jnp.int32, (_CHUNK, _LANES), 0)
    lane = jax.lax.broadcasted_iota(jnp.int32, (_CHUNK, _LANES), 1)
    row = p * _LANES + lane
    a = a_ref[0]
    unb = lambda b: jax.lax.bitcast_convert_type(b, jnp.uint32) ^ jnp.uint32(
        0x80000000)

    @pl.when(p < tc_progs)
    def _struct():
        sv, si = _tail_scan(k1s, k2s, row * STRUC_N + c_loc, c_loc, xs,
                            STRUC_N, 520)
        tok_s = _finish(unb(cand_b[0:1, :]), unb(cand_b[1:2, :]), unb(sv),
                        si, xs, a, STRUC_MASK)
        out_ref[0, 0:1, :] = tok_s

    qv, qi = _tail_scan(k1q, k2q, row * SEQ_N + c_loc, c_loc, xq,
                        SEQ_N, 40)
    tok_q = _finish(unb(cand_b[2:3, :]), unb(cand_b[3:4, :]), unb(qv), qi,
                    xq, a, SEQ_MASK)
    out_ref[0, 1:2, :] = tok_q
    out_ref[0, 2:8, :] = jnp.zeros((6, _LANES), jnp.int32)


def _gumbel_from_bits(bits):
    tiny = jnp.float32(jnp.finfo(jnp.float32).tiny)
    fb = (bits >> jnp.uint32(9)) | jnp.uint32(0x3F800000)
    floats = jax.lax.bitcast_convert_type(fb, jnp.float32) - jnp.float32(1.0)
    u = jnp.maximum(tiny, floats * (jnp.float32(1.0) - tiny) + tiny)
    return -jnp.log(-jnp.log(u))


def _finish(b_x0, b_mask, b_eps, i_eps, x_flat, a_flat, mask_idx):
    eq = x_flat == mask_idx
    one_minus_a = jnp.float32(1.0) - a_flat
    p_x0 = a_flat + jnp.where(eq, one_minus_a, jnp.float32(0.0))
    p_m = jnp.where(eq, a_flat + one_minus_a, one_minus_a)
    v1 = _gumbel_from_bits(b_x0) + jnp.log(p_x0 + EPS)
    v2 = _gumbel_from_bits(b_mask) + jnp.log(p_m + EPS)
    v3 = _gumbel_from_bits(b_eps) + jnp.log(jnp.float32(0.0) + EPS)
    i1 = x_flat
    i2 = jnp.full_like(x_flat, mask_idx)
    best_v, best_i = v1, i1
    upd = (v2 > best_v) | ((v2 == best_v) & (i2 < best_i))
    best_v = jnp.where(upd, v2, best_v)
    best_i = jnp.where(upd, i2, best_i)
    upd = (v3 > best_v) | ((v3 == best_v) & (i_eps < best_i))
    best_i = jnp.where(upd, i_eps, best_i)
    return best_i


def kernel(structure, sequence, t):
    t_idx = jnp.arange(T + 1, dtype=jnp.float32)
    beta = 1.0 / (T - t_idx + 1.0)
    alpha = jnp.cumprod(1.0 - beta)
    key = jax.random.key(42)
    ks, kq = jax.random.split(key)
    keys = jnp.concatenate([jax.random.key_data(ks),
                            jax.random.key_data(kq)]).astype(jnp.int32)
    B, L = structure.shape
    rows = B * L
    grid = rows // _LANES
    tc_rows = rows - _SC_ROWS
    tc_progs = tc_rows // _LANES
    xs = structure.reshape(grid, 1, _LANES).astype(jnp.int32)
    xq = sequence.reshape(grid, 1, _LANES).astype(jnp.int32)
    a_flat = jnp.repeat(alpha[t], L)
    a_in = a_flat.reshape(grid, 1, _LANES)
    x_struct_flat = structure.reshape(rows).astype(jnp.int32)
    sc_out = _sc_struct(x_struct_flat[tc_rows:], tc_rows)
    out = pl.pallas_call(
        functools.partial(_both_body, tc_progs=tc_progs),
        grid=(grid,),
        in_specs=[
            pl.BlockSpec(memory_space=pltpu.SMEM),
            pl.BlockSpec((1, 1, _LANES), lambda p: (p, 0, 0)),
            pl.BlockSpec((1, 1, _LANES), lambda p: (p, 0, 0)),
            pl.BlockSpec((1, 1, _LANES), lambda p: (p, 0, 0)),
        ],
        out_specs=pl.BlockSpec((1, 8, _LANES), lambda p: (p, 0, 0)),
        out_shape=jax.ShapeDtypeStruct((grid, 8, _LANES), jnp.int32),
        compiler_params=pltpu.CompilerParams(
            dimension_semantics=("parallel",)),
    )(keys, xs, xq, a_in)
    unbias = lambda b: jax.lax.bitcast_convert_type(b, jnp.uint32) ^ jnp.uint32(
        0x80000000)
    tok_sc = _finish(unbias(sc_out[0:_SC_ROWS]),
                     unbias(sc_out[_SC_ROWS:2 * _SC_ROWS]),
                     unbias(sc_out[2 * _SC_ROWS:3 * _SC_ROWS]),
                     sc_out[3 * _SC_ROWS:],
                     x_struct_flat[tc_rows:], a_flat[tc_rows:], STRUC_MASK)
    tok_s = jnp.concatenate(
        [out[:tc_progs, 0, :].reshape(tc_rows), tok_sc]).reshape(B, L)
    return (tok_s, out[:, 1, :].reshape(B, L), t)


# speedup vs baseline: 1.1085x; 1.0170x over previous
import functools

import jax
import jax.numpy as jnp
import numpy as np
from jax import lax
from jax.experimental import pallas as pl
from jax.experimental.pallas import tpu as pltpu
from jax.experimental.pallas import tpu_sc as plsc

T = 500
STRUC_N = 517
SEQ_N = 33
STRUC_MASK = 516
SEQ_MASK = 32
EPS = 1e-10
_NEG = np.int32(-(2 ** 31))
_BIG = np.int32(2 ** 30)
_LANES = 128
_CHUNK = 40

def _i32(x):
    return int(np.uint32(x & 0xFFFFFFFF).astype(np.int32))

_KS_S = (1832780943, 270669613)
_KS_Q = (64467757, _i32(2916123636))

_SC_ROWS = 24576
_NW = 32
_RW = _SC_ROWS // _NW
_GPI = 8


def _sc_keyset(k1, k2):
    ks2 = _i32(k1 ^ k2 ^ 0x1BD11BDA)
    return (_i32(k1), _i32(k2), ks2)


def _sc_tf(x1, ks):
    i32 = np.int32
    x0 = jnp.zeros_like(x1) + i32(ks[0])
    x1 = x1 + i32(ks[1])
    rot = (13, 15, 26, 6, 17, 29, 16, 24)
    rounds = (rot[0:4], rot[4:8], rot[0:4], rot[4:8], rot[0:4])
    for i, chunk in enumerate(rounds):
        for r in chunk:
            x0 = x0 + x1
            x1 = lax.shift_left(x1, i32(r)) | lax.shift_right_logical(
                x1, i32(32 - r))
            x1 = x0 ^ x1
        x0 = x0 + i32(ks[(i + 1) % 3])
        x1 = x1 + i32(_i32(ks[(i + 2) % 3] + (i + 1)))
    return x0 ^ x1 ^ i32(-(2 ** 31))


def _sc_struct(x_sc, row0):
    ks = _sc_keyset(*_KS_S)
    mesh = plsc.VectorSubcoreMesh(core_axis_name="c", subcore_axis_name="s")

    @functools.partial(
        pl.kernel, mesh=mesh,
        out_type=jax.ShapeDtypeStruct((4 * _SC_ROWS,), jnp.int32),
        scratch_types=[
            pltpu.VMEM((_RW,), jnp.int32),
            pltpu.VMEM((_RW,), jnp.int32),
            pltpu.VMEM((_RW,), jnp.int32),
            pltpu.VMEM((_RW,), jnp.int32),
            pltpu.VMEM((_RW,), jnp.int32),
        ],
    )
    def sck(x_hbm, out_hbm, xv, sx0, smk, sav, sai):
        wid = lax.axis_index("s") * 2 + lax.axis_index("c")
        base = wid * _RW
        pltpu.sync_copy(x_hbm.at[pl.ds(base, _RW)], xv)
        iota = lax.iota(jnp.int32, 16)
        for q in range(_RW // (16 * _GPI)):
            off = q * 16 * _GPI
            x0v, ib = [], []
            for j in range(_GPI):
                xj = xv[pl.ds(off + j * 16, 16)]
                rowv = row0 + base + off + j * 16 + iota
                ibj = rowv * np.int32(STRUC_N)
                x0v.append(xj)
                ib.append(ibj)
                sx0[pl.ds(off + j * 16, 16)] = _sc_tf(ibj + xj, ks)
                smk[pl.ds(off + j * 16, 16)] = _sc_tf(
                    ibj + np.int32(STRUC_MASK), ks)

            def cls_body(c, carry):
                avs, ais = carry
                navs, nais = [], []
                for j in range(_GPI):
                    b = _sc_tf(ib[j] + c, ks)
                    be = jnp.where(x0v[j] == c, _NEG, b)
                    upd = be > avs[j]
                    navs.append(jnp.maximum(avs[j], be))
                    nais.append(jnp.where(upd, c, ais[j]))
                return tuple(navs), tuple(nais)

            init = (tuple(jnp.full((16,), _NEG, jnp.int32)
                          for _ in range(_GPI)),
                    tuple(jnp.full((16,), _BIG, jnp.int32)
                          for _ in range(_GPI)))
            avs, ais = lax.fori_loop(0, STRUC_N - 1, cls_body, init)
            for j in range(_GPI):
                sav[pl.ds(off + j * 16, 16)] = avs[j]
                sai[pl.ds(off + j * 16, 16)] = ais[j]
        pltpu.sync_copy(sx0, out_hbm.at[pl.ds(0 * _SC_ROWS + base, _RW)])
        pltpu.sync_copy(smk, out_hbm.at[pl.ds(1 * _SC_ROWS + base, _RW)])
        pltpu.sync_copy(sav, out_hbm.at[pl.ds(2 * _SC_ROWS + base, _RW)])
        pltpu.sync_copy(sai, out_hbm.at[pl.ds(3 * _SC_ROWS + base, _RW)])

    return sck(x_sc)


def _threefry_biased(k1, k2, x1):
    ks2 = k1 ^ k2 ^ jnp.uint32(0x1BD11BDA)
    ks = (k1, k2, ks2)
    x0 = jnp.zeros_like(x1) + k1
    x1 = x1 + k2
    rot = (13, 15, 26, 6, 17, 29, 16, 24)
    rounds = (rot[0:4], rot[4:8], rot[0:4], rot[4:8], rot[0:4])
    for i, chunk in enumerate(rounds):
        for r in chunk:
            x0 = x0 + x1
            x1 = (x1 << jnp.uint32(r)) | (x1 >> jnp.uint32(32 - r))
            x1 = x0 ^ x1
        x0 = x0 + ks[(i + 1) % 3]
        x1 = x1 + ks[(i + 2) % 3] + jnp.uint32(i + 1)
    return jax.lax.bitcast_convert_type(x0 ^ x1 ^ jnp.uint32(0x80000000),
                                        jnp.int32)


def _combine(av, ai, bv, bi):
    take = (bv > av) | ((bv == av) & (bi < ai))
    return jnp.maximum(av, bv), jnp.where(take, bi, ai)


def _tail_scan(k1, k2, ibase, c_loc, x0, n_cls, n_pad):
    acc_v = jnp.full((_CHUNK, _LANES), _NEG, jnp.int32)
    acc_i = jnp.full((_CHUNK, _LANES), _BIG, jnp.int32)
    for c0 in range(0, n_pad, _CHUNK):
        biased = _threefry_biased(k1, k2, (ibase + c0).astype(jnp.uint32))
        excl = c_loc == (x0 - c0)
        thr = n_cls - 1 - c0
        if thr < _CHUNK:
            excl = excl | (c_loc >= thr)
        b_eps = jnp.where(excl, _NEG, biased)
        upd = b_eps > acc_v
        acc_v = jnp.maximum(acc_v, b_eps)
        acc_i = jnp.where(upd, c_loc + c0, acc_i)
    n = _CHUNK
    while n > 1:
        h = n // 2
        mv, mi = _combine(acc_v[:h], acc_i[:h], acc_v[h:2 * h],
                          acc_i[h:2 * h])
        if n % 2:
            mv = jnp.concatenate([mv, acc_v[2 * h:n]], axis=0)
            mi = jnp.concatenate([mi, acc_i[2 * h:n]], axis=0)
        acc_v, acc_i = mv, mi
        n = h + (n % 2)
    return acc_v, acc_i


def _both_body(keys_ref, xs_ref, xq_ref, a_ref, out_ref, *, tc_progs):
    p = pl.program_id(0)
    u32 = lambda v: jax.lax.bitcast_convert_type(v, jnp.uint32)
    k1s, k2s = u32(keys_ref[0]), u32(keys_ref[1])
    k1q, k2q = u32(keys_ref[2]), u32(keys_ref[3])
    xs = xs_ref[0]
    xq = xq_ref[0]
    lane1 = jax.lax.broadcasted_iota(jnp.int32, (1, _LANES), 1)
    row1 = p * _LANES + lane1
    ibs1 = row1 * STRUC_N
    ibq1 = row1 * SEQ_N

    cand_i = jnp.concatenate(
        [ibs1 + xs, ibs1 + STRUC_MASK, ibq1 + xq, ibq1 + SEQ_MASK,
         jnp.zeros((4, _LANES), jnp.int32)], axis=0)
    srow = jax.lax.broadcasted_iota(jnp.int32, (8, 1), 0) < 2
    ck1 = jnp.where(srow, k1s, k1q)
    ck2 = jnp.where(srow, k2s, k2q)
    cand_b = _threefry_biased(ck1, ck2, cand_i.astype(jnp.uint32))

    c_loc = jax.lax.broadcasted_iota(jnp.int32, (_CHUNK, _LANES), 0)
    lane = jax.lax.broadcasted_iota(jnp.int32, (_CHUNK, _LANES), 1)
    row = p * _LANES + lane
    a = a_ref[0]
    unb = lambda b: jax.lax.bitcast_convert_type(b, jnp.uint32) ^ jnp.uint32(
        0x80000000)

    @pl.when(p < tc_progs)
    def _struct():
        sv, si = _tail_scan(k1s, k2s, row * STRUC_N + c_loc, c_loc, xs,
                            STRUC_N, 520)
        tok_s = _finish(unb(cand_b[0:1, :]), unb(cand_b[1:2, :]), unb(sv),
                        si, xs, a, STRUC_MASK)
        out_ref[0, 0:1, :] = tok_s

    qv, qi = _tail_scan(k1q, k2q, row * SEQ_N + c_loc, c_loc, xq,
                        SEQ_N, 40)
    tok_q = _finish(unb(cand_b[2:3, :]), unb(cand_b[3:4, :]), unb(qv), qi,
                    xq, a, SEQ_MASK)
    out_ref[0, 1:2, :] = tok_q
    out_ref[0, 2:8, :] = jnp.zeros((6, _LANES), jnp.int32)


def _gumbel_from_bits(bits):
    tiny = jnp.float32(jnp.finfo(jnp.float32).tiny)
    fb = (bits >> jnp.uint32(9)) | jnp.uint32(0x3F800000)
    floats = jax.lax.bitcast_convert_type(fb, jnp.float32) - jnp.float32(1.0)
    u = jnp.maximum(tiny, floats * (jnp.float32(1.0) - tiny) + tiny)
    return -jnp.log(-jnp.log(u))


def _finish(b_x0, b_mask, b_eps, i_eps, x_flat, a_flat, mask_idx):
    eq = x_flat == mask_idx
    one_minus_a = jnp.float32(1.0) - a_flat
    p_x0 = a_flat + jnp.where(eq, one_minus_a, jnp.float32(0.0))
    p_m = jnp.where(eq, a_flat + one_minus_a, one_minus_a)
    v1 = _gumbel_from_bits(b_x0) + jnp.log(p_x0 + EPS)
    v2 = _gumbel_from_bits(b_mask) + jnp.log(p_m + EPS)
    v3 = _gumbel_from_bits(b_eps) + jnp.log(jnp.float32(0.0) + EPS)
    i1 = x_flat
    i2 = jnp.full_like(x_flat, mask_idx)
    best_v, best_i = v1, i1
    upd = (v2 > best_v) | ((v2 == best_v) & (i2 < best_i))
    best_v = jnp.where(upd, v2, best_v)
    best_i = jnp.where(upd, i2, best_i)
    upd = (v3 > best_v) | ((v3 == best_v) & (i_eps < best_i))
    best_i = jnp.where(upd, i_eps, best_i)
    return best_i


def kernel(structure, sequence, t):
    t_idx = jnp.arange(T + 1, dtype=jnp.float32)
    beta = 1.0 / (T - t_idx + 1.0)
    alpha = jnp.cumprod(1.0 - beta)
    key = jax.random.key(42)
    ks, kq = jax.random.split(key)
    keys = jnp.concatenate([jax.random.key_data(ks),
                            jax.random.key_data(kq)]).astype(jnp.int32)
    B, L = structure.shape
    rows = B * L
    grid = rows // _LANES
    tc_rows = rows - _SC_ROWS
    tc_progs = tc_rows // _LANES
    xs = structure.reshape(grid, 1, _LANES).astype(jnp.int32)
    xq = sequence.reshape(grid, 1, _LANES).astype(jnp.int32)
    a_flat = jnp.repeat(alpha[t], L)
    a_in = a_flat.reshape(grid, 1, _LANES)
    x_struct_flat = structure.reshape(rows).astype(jnp.int32)
    sc_out = _sc_struct(x_struct_flat[tc_rows:], tc_rows)
    out = pl.pallas_call(
        functools.partial(_both_body, tc_progs=tc_progs),
        grid=(grid,),
        in_specs=[
            pl.BlockSpec(memory_space=pltpu.SMEM),
            pl.BlockSpec((1, 1, _LANES), lambda p: (p, 0, 0)),
            pl.BlockSpec((1, 1, _LANES), lambda p: (p, 0, 0)),
            pl.BlockSpec((1, 1, _LANES), lambda p: (p, 0, 0)),
        ],
        out_specs=pl.BlockSpec((1, 8, _LANES), lambda p: (p, 0, 0)),
        out_shape=jax.ShapeDtypeStruct((grid, 8, _LANES), jnp.int32),
        compiler_params=pltpu.CompilerParams(
            dimension_semantics=("parallel",)),
    )(keys, xs, xq, a_in)
    unbias = lambda b: jax.lax.bitcast_convert_type(b, jnp.uint32) ^ jnp.uint32(
        0x80000000)
    tok_sc = _finish(unbias(sc_out[0:_SC_ROWS]),
                     unbias(sc_out[_SC_ROWS:2 * _SC_ROWS]),
                     unbias(sc_out[2 * _SC_ROWS:3 * _SC_ROWS]),
                     sc_out[3 * _SC_ROWS:],
                     x_struct_flat[tc_rows:], a_flat[tc_rows:], STRUC_MASK)
    tok_s = jnp.concatenate(
        [out[:tc_progs, 0, :].reshape(tc_rows), tok_sc]).reshape(B, L)
    return (tok_s, out[:, 1, :].reshape(B, L), t)


# speedup vs baseline: 1.1088x; 1.0002x over previous
import functools

import jax
import jax.numpy as jnp
import numpy as np
from jax import lax
from jax.experimental import pallas as pl
from jax.experimental.pallas import tpu as pltpu
from jax.experimental.pallas import tpu_sc as plsc

T = 500
STRUC_N = 517
SEQ_N = 33
STRUC_MASK = 516
SEQ_MASK = 32
EPS = 1e-10
_NEG = np.int32(-(2 ** 31))
_BIG = np.int32(2 ** 30)
_LANES = 256
_CHUNK = 40

def _i32(x):
    return int(np.uint32(x & 0xFFFFFFFF).astype(np.int32))

_KS_S = (1832780943, 270669613)
_KS_Q = (64467757, _i32(2916123636))

_SC_ROWS = 24576
_NW = 32
_RW = _SC_ROWS // _NW
_GPI = 8


def _sc_keyset(k1, k2):
    ks2 = _i32(k1 ^ k2 ^ 0x1BD11BDA)
    return (_i32(k1), _i32(k2), ks2)


def _sc_tf(x1, ks):
    i32 = np.int32
    x0 = jnp.zeros_like(x1) + i32(ks[0])
    x1 = x1 + i32(ks[1])
    rot = (13, 15, 26, 6, 17, 29, 16, 24)
    rounds = (rot[0:4], rot[4:8], rot[0:4], rot[4:8], rot[0:4])
    for i, chunk in enumerate(rounds):
        for r in chunk:
            x0 = x0 + x1
            x1 = lax.shift_left(x1, i32(r)) | lax.shift_right_logical(
                x1, i32(32 - r))
            x1 = x0 ^ x1
        x0 = x0 + i32(ks[(i + 1) % 3])
        x1 = x1 + i32(_i32(ks[(i + 2) % 3] + (i + 1)))
    return x0 ^ x1 ^ i32(-(2 ** 31))


def _sc_struct(x_sc, row0):
    ks = _sc_keyset(*_KS_S)
    mesh = plsc.VectorSubcoreMesh(core_axis_name="c", subcore_axis_name="s")

    @functools.partial(
        pl.kernel, mesh=mesh,
        out_type=jax.ShapeDtypeStruct((4 * _SC_ROWS,), jnp.int32),
        scratch_types=[
            pltpu.VMEM((_RW,), jnp.int32),
            pltpu.VMEM((_RW,), jnp.int32),
            pltpu.VMEM((_RW,), jnp.int32),
            pltpu.VMEM((_RW,), jnp.int32),
            pltpu.VMEM((_RW,), jnp.int32),
        ],
    )
    def sck(x_hbm, out_hbm, xv, sx0, smk, sav, sai):
        wid = lax.axis_index("s") * 2 + lax.axis_index("c")
        base = wid * _RW
        pltpu.sync_copy(x_hbm.at[pl.ds(base, _RW)], xv)
        iota = lax.iota(jnp.int32, 16)
        for q in range(_RW // (16 * _GPI)):
            off = q * 16 * _GPI
            x0v, ib = [], []
            for j in range(_GPI):
                xj = xv[pl.ds(off + j * 16, 16)]
                rowv = row0 + base + off + j * 16 + iota
                ibj = rowv * np.int32(STRUC_N)
                x0v.append(xj)
                ib.append(ibj)
                sx0[pl.ds(off + j * 16, 16)] = _sc_tf(ibj + xj, ks)
                smk[pl.ds(off + j * 16, 16)] = _sc_tf(
                    ibj + np.int32(STRUC_MASK), ks)

            def cls_body(c, carry):
                avs, ais = carry
                navs, nais = [], []
                for j in range(_GPI):
                    b = _sc_tf(ib[j] + c, ks)
                    be = jnp.where(x0v[j] == c, _NEG, b)
                    upd = be > avs[j]
                    navs.append(jnp.maximum(avs[j], be))
                    nais.append(jnp.where(upd, c, ais[j]))
                return tuple(navs), tuple(nais)

            init = (tuple(jnp.full((16,), _NEG, jnp.int32)
                          for _ in range(_GPI)),
                    tuple(jnp.full((16,), _BIG, jnp.int32)
                          for _ in range(_GPI)))
            avs, ais = lax.fori_loop(0, STRUC_N - 1, cls_body, init)
            for j in range(_GPI):
                sav[pl.ds(off + j * 16, 16)] = avs[j]
                sai[pl.ds(off + j * 16, 16)] = ais[j]
        pltpu.sync_copy(sx0, out_hbm.at[pl.ds(0 * _SC_ROWS + base, _RW)])
        pltpu.sync_copy(smk, out_hbm.at[pl.ds(1 * _SC_ROWS + base, _RW)])
        pltpu.sync_copy(sav, out_hbm.at[pl.ds(2 * _SC_ROWS + base, _RW)])
        pltpu.sync_copy(sai, out_hbm.at[pl.ds(3 * _SC_ROWS + base, _RW)])

    return sck(x_sc)


def _threefry_biased(k1, k2, x1):
    ks2 = k1 ^ k2 ^ jnp.uint32(0x1BD11BDA)
    ks = (k1, k2, ks2)
    x0 = jnp.zeros_like(x1) + k1
    x1 = x1 + k2
    rot = (13, 15, 26, 6, 17, 29, 16, 24)
    rounds = (rot[0:4], rot[4:8], rot[0:4], rot[4:8], rot[0:4])
    for i, chunk in enumerate(rounds):
        for r in chunk:
            x0 = x0 + x1
            x1 = (x1 << jnp.uint32(r)) | (x1 >> jnp.uint32(32 - r))
            x1 = x0 ^ x1
        x0 = x0 + ks[(i + 1) % 3]
        x1 = x1 + ks[(i + 2) % 3] + jnp.uint32(i + 1)
    return jax.lax.bitcast_convert_type(x0 ^ x1 ^ jnp.uint32(0x80000000),
                                        jnp.int32)


def _combine(av, ai, bv, bi):
    take = (bv > av) | ((bv == av) & (bi < ai))
    return jnp.maximum(av, bv), jnp.where(take, bi, ai)


def _tail_scan(k1, k2, ibase, c_loc, x0, n_cls, n_pad):
    acc_v = jnp.full((_CHUNK, _LANES), _NEG, jnp.int32)
    acc_i = jnp.full((_CHUNK, _LANES), _BIG, jnp.int32)
    for c0 in range(0, n_pad, _CHUNK):
        biased = _threefry_biased(k1, k2, (ibase + c0).astype(jnp.uint32))
        excl = c_loc == (x0 - c0)
        thr = n_cls - 1 - c0
        if thr < _CHUNK:
            excl = excl | (c_loc >= thr)
        b_eps = jnp.where(excl, _NEG, biased)
        upd = b_eps > acc_v
        acc_v = jnp.maximum(acc_v, b_eps)
        acc_i = jnp.where(upd, c_loc + c0, acc_i)
    n = _CHUNK
    while n > 1:
        h = n // 2
        mv, mi = _combine(acc_v[:h], acc_i[:h], acc_v[h:2 * h],
                          acc_i[h:2 * h])
        if n % 2:
            mv = jnp.concatenate([mv, acc_v[2 * h:n]], axis=0)
            mi = jnp.concatenate([mi, acc_i[2 * h:n]], axis=0)
        acc_v, acc_i = mv, mi
        n = h + (n % 2)
    return acc_v, acc_i


def _both_body(keys_ref, xs_ref, xq_ref, a_ref, out_ref, *, tc_progs):
    p = pl.program_id(0)
    u32 = lambda v: jax.lax.bitcast_convert_type(v, jnp.uint32)
    k1s, k2s = u32(keys_ref[0]), u32(keys_ref[1])
    k1q, k2q = u32(keys_ref[2]), u32(keys_ref[3])
    xs = xs_ref[0]
    xq = xq_ref[0]
    lane1 = jax.lax.broadcasted_iota(jnp.int32, (1, _LANES), 1)
    row1 = p * _LANES + lane1
    ibs1 = row1 * STRUC_N
    ibq1 = row1 * SEQ_N

    cand_i = jnp.concatenate(
        [ibs1 + xs, ibs1 + STRUC_MASK, ibq1 + xq, ibq1 + SEQ_MASK,
         jnp.zeros((4, _LANES), jnp.int32)], axis=0)
    srow = jax.lax.broadcasted_iota(jnp.int32, (8, 1), 0) < 2
    ck1 = jnp.where(srow, k1s, k1q)
    ck2 = jnp.where(srow, k2s, k2q)
    cand_b = _threefry_biased(ck1, ck2, cand_i.astype(jnp.uint32))

    c_loc = jax.lax.broadcasted_iota(jnp.int32, (_CHUNK, _LANES), 0)
    lane = jax.lax.broadcasted_iota(jnp.int32, (_CHUNK, _LANES), 1)
    row = p * _LANES + lane
    a = a_ref[0]
    unb = lambda b: jax.lax.bitcast_convert_type(b, jnp.uint32) ^ jnp.uint32(
        0x80000000)

    @pl.when(p < tc_progs)
    def _struct():
        sv, si = _tail_scan(k1s, k2s, row * STRUC_N + c_loc, c_loc, xs,
                            STRUC_N, 520)
        tok_s = _finish(unb(cand_b[0:1, :]), unb(cand_b[1:2, :]), unb(sv),
                        si, xs, a, STRUC_MASK)
        out_ref[0, 0:1, :] = tok_s

    qv, qi = _tail_scan(k1q, k2q, row * SEQ_N + c_loc, c_loc, xq,
                        SEQ_N, 40)
    tok_q = _finish(unb(cand_b[2:3, :]), unb(cand_b[3:4, :]), unb(qv), qi,
                    xq, a, SEQ_MASK)
    out_ref[0, 1:2, :] = tok_q
    out_ref[0, 2:8, :] = jnp.zeros((6, _LANES), jnp.int32)


def _gumbel_from_bits(bits):
    tiny = jnp.float32(jnp.finfo(jnp.float32).tiny)
    fb = (bits >> jnp.uint32(9)) | jnp.uint32(0x3F800000)
    floats = jax.lax.bitcast_convert_type(fb, jnp.float32) - jnp.float32(1.0)
    u = jnp.maximum(tiny, floats * (jnp.float32(1.0) - tiny) + tiny)
    return -jnp.log(-jnp.log(u))


def _finish(b_x0, b_mask, b_eps, i_eps, x_flat, a_flat, mask_idx):
    eq = x_flat == mask_idx
    one_minus_a = jnp.float32(1.0) - a_flat
    p_x0 = a_flat + jnp.where(eq, one_minus_a, jnp.float32(0.0))
    p_m = jnp.where(eq, a_flat + one_minus_a, one_minus_a)
    v1 = _gumbel_from_bits(b_x0) + jnp.log(p_x0 + EPS)
    v2 = _gumbel_from_bits(b_mask) + jnp.log(p_m + EPS)
    v3 = _gumbel_from_bits(b_eps) + jnp.log(jnp.float32(0.0) + EPS)
    i1 = x_flat
    i2 = jnp.full_like(x_flat, mask_idx)
    best_v, best_i = v1, i1
    upd = (v2 > best_v) | ((v2 == best_v) & (i2 < best_i))
    best_v = jnp.where(upd, v2, best_v)
    best_i = jnp.where(upd, i2, best_i)
    upd = (v3 > best_v) | ((v3 == best_v) & (i_eps < best_i))
    best_i = jnp.where(upd, i_eps, best_i)
    return best_i


def kernel(structure, sequence, t):
    t_idx = jnp.arange(T + 1, dtype=jnp.float32)
    beta = 1.0 / (T - t_idx + 1.0)
    alpha = jnp.cumprod(1.0 - beta)
    key = jax.random.key(42)
    ks, kq = jax.random.split(key)
    keys = jnp.concatenate([jax.random.key_data(ks),
                            jax.random.key_data(kq)]).astype(jnp.int32)
    B, L = structure.shape
    rows = B * L
    grid = rows // _LANES
    tc_rows = rows - _SC_ROWS
    tc_progs = tc_rows // _LANES
    xs = structure.reshape(grid, 1, _LANES).astype(jnp.int32)
    xq = sequence.reshape(grid, 1, _LANES).astype(jnp.int32)
    a_flat = jnp.repeat(alpha[t], L)
    a_in = a_flat.reshape(grid, 1, _LANES)
    x_struct_flat = structure.reshape(rows).astype(jnp.int32)
    sc_out = _sc_struct(x_struct_flat[tc_rows:], tc_rows)
    out = pl.pallas_call(
        functools.partial(_both_body, tc_progs=tc_progs),
        grid=(grid,),
        in_specs=[
            pl.BlockSpec(memory_space=pltpu.SMEM),
            pl.BlockSpec((1, 1, _LANES), lambda p: (p, 0, 0)),
            pl.BlockSpec((1, 1, _LANES), lambda p: (p, 0, 0)),
            pl.BlockSpec((1, 1, _LANES), lambda p: (p, 0, 0)),
        ],
        out_specs=pl.BlockSpec((1, 8, _LANES), lambda p: (p, 0, 0)),
        out_shape=jax.ShapeDtypeStruct((grid, 8, _LANES), jnp.int32),
        compiler_params=pltpu.CompilerParams(
            dimension_semantics=("parallel",)),
    )(keys, xs, xq, a_in)
    unbias = lambda b: jax.lax.bitcast_convert_type(b, jnp.uint32) ^ jnp.uint32(
        0x80000000)
    tok_sc = _finish(unbias(sc_out[0:_SC_ROWS]),
                     unbias(sc_out[_SC_ROWS:2 * _SC_ROWS]),
                     unbias(sc_out[2 * _SC_ROWS:3 * _SC_ROWS]),
                     sc_out[3 * _SC_ROWS:],
                     x_struct_flat[tc_rows:], a_flat[tc_rows:], STRUC_MASK)
    tok_s = jnp.concatenate(
        [out[:tc_progs, 0, :].reshape(tc_rows), tok_sc]).reshape(B, L)
    return (tok_s, out[:, 1, :].reshape(B, L), t)
